# baseline (speedup 1.0000x reference)
import math

import jax
import jax.numpy as jnp
from jax.experimental import pallas as pl
from jax.experimental.pallas import tpu as pltpu

W_DAYS, S_SLOTS = 7, 288
N, E = 64, 256
D, DR, DC, K_SEQ = 64, 32, 32, 12
B = W_DAYS * S_SLOTS
SB = 144
F32 = jnp.float32

_INTERPRET = False


def _dot(a, b):
    return jax.lax.dot_general(a, b, (((1,), (0,)), ((), ())),
                               preferred_element_type=F32)


def _cmm(a, b):
    return jax.lax.dot_general(a, b, (((2,), (0,)), ((), ())),
                               preferred_element_type=F32)


def _attn_p(z, M, wad, C_rep, scale):
    nh = wad.shape[1]
    e = _cmm(z, M)
    ad_cols = _cmm(z, wad)
    ad_ = jnp.swapaxes(ad_cols, 1, 2).reshape(z.shape[0], 1, nh * N)
    e = e + ad_
    e = jnp.maximum(e, 0.2 * e)
    p = C_rep[None] * jnp.exp(e)
    rden = scale / (jnp.sum(p, axis=1) + 1e-16)
    return p * rden[:, None, :]


def _krn(H_ref, feat_ref, seq_ref, edge_ref,
         wi_r, wi_z, wi_n, wh_r, wh_z, wh_n,
         bi_r, bi_z, bi_n, bh_r, bh_z, bh_n,
         rl1_w, rl1_b, rl2_w, rl2_b,
         cl1_w, cl1_b, cl2_w, cl2_b,
         d1H, d1R, d1C, d1_b, d2_w, d2_b, g1H, g1R, g1C, g1_b,
         gat1_W01, gat1_M, gat1_wad, gat1_b,
         gat2_W, gat2_M, gat2_wad, gat2_b,
         s1_w, s1_b, s2_col, s2_b,
         delta_ref, Had_ref, pred_ref,
         C_scr, C2_scr, rd_scr, rg_scr):
    pid = pl.program_id(0)

    @pl.when(pid == 0)
    def _init():
        ids = jax.lax.broadcasted_iota(jnp.int32, (N, E), 0)
        s_oh = (ids == edge_ref[0:1, :]).astype(F32)
        d_oh = (ids == edge_ref[1:2, :]).astype(F32)
        C_T = jax.lax.dot_general(s_oh, d_oh, (((1,), (1,)), ((), ())),
                                  preferred_element_type=F32)
        ri = jax.lax.broadcasted_iota(jnp.int32, (N, N), 0)
        ci = jax.lax.broadcasted_iota(jnp.int32, (N, N), 1)
        C_T = C_T + (ri == ci).astype(F32)
        C_scr[...] = C_T
        C2_scr[...] = jnp.concatenate([C_T, C_T], axis=1)

        h = jnp.zeros((N, DR), F32)
        for t in range(K_SEQ):
            x_t = seq_ref[:, t:t + 1]
            r = jax.nn.sigmoid(x_t * wi_r[...] + bi_r[...]
                               + _dot(h, wh_r[...]) + bh_r[...])
            z = jax.nn.sigmoid(x_t * wi_z[...] + bi_z[...]
                               + _dot(h, wh_z[...]) + bh_z[...])
            nn_ = jnp.tanh(x_t * wi_n[...] + bi_n[...]
                           + r * (_dot(h, wh_n[...]) + bh_n[...]))
            h = (1.0 - z) * nn_ + z * h
        x = jnp.maximum(_dot(h, rl1_w[...]) + rl1_b[...], 0.0)
        rec = _dot(x, rl2_w[...]) + rl2_b[...]
        rd_scr[...] = _dot(rec, d1R[...]) + d1_b[...]
        rg_scr[...] = _dot(rec, g1R[...]) + g1_b[...]

    C_T = C_scr[...]
    C_T2 = C2_scr[...]
    H3 = H_ref[...]
    H2 = H3.reshape(SB * N, D)

    feat = feat_ref[...]
    cal = jnp.maximum(_dot(feat, cl1_w[...]) + cl1_b[...], 0.0)
    cal = _dot(cal, cl2_w[...]) + cl2_b[...]

    cal_d1 = _dot(cal, d1C[...])[:, None, :]
    cal_g1 = _dot(cal, g1C[...])[:, None, :]
    pre_d = _dot(H2, d1H[...]).reshape(SB, N, D) + rd_scr[...][None] + cal_d1
    pre_g = _dot(H2, g1H[...]).reshape(SB, N, D) + rg_scr[...][None] + cal_g1
    hid = jnp.maximum(pre_d, 0.0).reshape(SB * N, D)
    delta_raw = _dot(hid, d2_w[...]) + d2_b[...]
    gate = jax.nn.sigmoid(pre_g.reshape(SB * N, D))
    delta0 = delta_raw * gate

    delta0_3 = delta0.reshape(SB, N, D)
    xh01 = _dot(delta0, gat1_W01[...]).reshape(SB, N, 2 * D)
    p01 = _attn_p(delta0_3, gat1_M[...], gat1_wad[...], C_T2, 0.5)
    p_v = jnp.concatenate([p01[:, :, :N], p01[:, :, N:]], axis=1)
    x_v = jnp.concatenate([xh01[:, :, :D], xh01[:, :, D:]], axis=1)
    agg1 = jax.lax.dot_general(p_v, x_v, (((1,), (1,)), ((0,), (0,))),
                               preferred_element_type=F32)
    x1 = jnp.maximum(agg1 + gat1_b[...][None], 0.0)

    xl2 = _dot(x1.reshape(SB * N, D), gat2_W[...]).reshape(SB, N, D)
    p2 = _attn_p(x1, gat2_M[...], gat2_wad[...], C_T, 1.0)
    o2 = jax.lax.dot_general(p2, xl2, (((1,), (1,)), ((0,), (0,))),
                             preferred_element_type=F32)
    delta = jnp.maximum(o2 + gat2_b[...][None], 0.0)

    H_ad = H3 + delta
    delta_ref[...] = delta
    Had_ref[...] = H_ad

    hh = jnp.maximum(_cmm(H_ad, s1_w[...]) + s1_b[...][None], 0.0)
    pred_ref[...] = _cmm(hh, s2_col[...]) + s2_b[...]


def kernel(H_base_bank, recent_speed_seq, edge_index, params):
    p = params
    H = H_base_bank.reshape(B, N, D)
    seq = recent_speed_seq[:, :, 0]

    wd = jnp.repeat(jnp.arange(7, dtype=F32), S_SLOTS)
    sl = jnp.tile(jnp.arange(S_SLOTS, dtype=F32), (7,))
    ta = (2.0 * math.pi / S_SLOTS) * sl
    wa = (2.0 * math.pi / 7.0) * wd
    cyc = jnp.stack([jnp.sin(ta), jnp.cos(ta), jnp.sin(wa), jnp.cos(wa)],
                    axis=-1)
    wk = ((wd == 5.0) | (wd == 6.0)).astype(F32)[:, None]
    feat = jnp.concatenate(
        [jnp.repeat(p['wd_emb'], S_SLOTS, axis=0),
         jnp.tile(p['slot_emb'], (7, 1)), cyc, wk], axis=-1)

    r2 = lambda a: a.reshape(1, -1)
    w_ih, w_hh = p['gru_W_ih'], p['gru_W_hh']
    b_ih, b_hh = p['gru_b_ih'], p['gru_b_hh']
    ins = [
        H, feat, seq, edge_index,
        w_ih[:DR].T, w_ih[DR:2 * DR].T, w_ih[2 * DR:].T,
        w_hh[:DR].T, w_hh[DR:2 * DR].T, w_hh[2 * DR:].T,
        r2(b_ih[:DR]), r2(b_ih[DR:2 * DR]), r2(b_ih[2 * DR:]),
        r2(b_hh[:DR]), r2(b_hh[DR:2 * DR]), r2(b_hh[2 * DR:]),
        p['rec_l1_w'], r2(p['rec_l1_b']), p['rec_l2_w'], r2(p['rec_l2_b']),
        p['cal_l1_w'], r2(p['cal_l1_b']), p['cal_l2_w'], r2(p['cal_l2_b']),
        p['d1_w'][:D], p['d1_w'][D:D + DR], p['d1_w'][D + DR:], r2(p['d1_b']),
        p['d2_w'], r2(p['d2_b']),
        p['g1_w'][:D], p['g1_w'][D:D + DR], p['g1_w'][D + DR:], r2(p['g1_b']),
        p['gat1_W'],
        jnp.concatenate(
            [jnp.broadcast_to((p['gat1_W'][:, h * D:(h + 1) * D]
                               @ p['gat1_as'][h])[:, None], (D, N))
             for h in (0, 1)], axis=1),
        jnp.stack([p['gat1_W'][:, :D] @ p['gat1_ad'][0],
                   p['gat1_W'][:, D:] @ p['gat1_ad'][1]], axis=1),
        r2(p['gat1_b']),
        p['gat2_W'],
        jnp.broadcast_to((p['gat2_W'] @ p['gat2_as'][0])[:, None], (D, N)),
        (p['gat2_W'] @ p['gat2_ad'][0])[:, None], r2(p['gat2_b']),
        p['s1_w'], r2(p['s1_b']), p['s2_w'], p['s2_b'].reshape(1, 1, 1),
    ]

    def const_spec(a):
        zeros = (0,) * a.ndim
        return pl.BlockSpec(a.shape, lambda i, z=zeros: z)

    in_specs = ([pl.BlockSpec((SB, N, D), lambda i: (i, 0, 0)),
                 pl.BlockSpec((SB, 29), lambda i: (i, 0))]
                + [const_spec(a) for a in ins[2:]])

    out_shape = (jax.ShapeDtypeStruct((B, N, D), F32),
                 jax.ShapeDtypeStruct((B, N, D), F32),
                 jax.ShapeDtypeStruct((B, N, 1), F32))
    out_specs = (pl.BlockSpec((SB, N, D), lambda i: (i, 0, 0)),
                 pl.BlockSpec((SB, N, D), lambda i: (i, 0, 0)),
                 pl.BlockSpec((SB, N, 1), lambda i: (i, 0, 0)))

    delta, H_ad, pred = pl.pallas_call(
        _krn,
        grid=(B // SB,),
        in_specs=in_specs,
        out_specs=out_specs,
        out_shape=out_shape,
        scratch_shapes=[pltpu.VMEM((N, N), F32), pltpu.VMEM((N, 2 * N), F32),
                        pltpu.VMEM((N, D), F32), pltpu.VMEM((N, D), F32)],
        interpret=_INTERPRET,
    )(*ins)

    return (delta.reshape(W_DAYS, S_SLOTS, N, D),
            H_ad.reshape(W_DAYS, S_SLOTS, N, D),
            pred.reshape(W_DAYS, S_SLOTS, N))


# speedup vs baseline: 1.0451x; 1.0451x over previous
import math

import jax
import jax.numpy as jnp
from jax.experimental import pallas as pl
from jax.experimental.pallas import tpu as pltpu

W_DAYS, S_SLOTS = 7, 288
N, E = 64, 256
D, DR, DC, K_SEQ = 64, 32, 32, 12
B = W_DAYS * S_SLOTS
SB = 144
F32 = jnp.float32

_INTERPRET = False


def _dot(a, b):
    return jax.lax.dot_general(a, b, (((1,), (0,)), ((), ())),
                               preferred_element_type=F32)


def _cmm(a, b):
    return jax.lax.dot_general(a, b, (((2,), (0,)), ((), ())),
                               preferred_element_type=F32)


def _attn_p(z, M, wad, C_rep, scale):
    nh = wad.shape[1]
    ad_rows = jnp.swapaxes(_cmm(z, wad), 1, 2)
    ps = []
    for h in range(nh):
        e = _cmm(z, M[:, h * N:(h + 1) * N])
        e = e + ad_rows[:, h:h + 1, :]
        e = jnp.maximum(e, 0.2 * e)
        p = C_rep[None] * jnp.exp(e)
        rden = scale / (jnp.sum(p, axis=1) + 1e-16)
        ps.append(p * rden[:, None, :])
    return jnp.concatenate(ps, axis=1) if nh > 1 else ps[0]


def _krn(H_ref, feat_ref, seq_ref, edge_ref,
         wi_r, wi_z, wi_n, wh_r, wh_z, wh_n,
         bi_r, bi_z, bi_n, bh_r, bh_z, bh_n,
         rl1_w, rl1_b, rl2_w, rl2_b,
         cl1_w, cl1_b, cl2_w, cl2_b,
         d1H, d1R, d1C, d1_b, d2_w, d2_b, g1H, g1R, g1C, g1_b,
         gat1_W01, gat1_M, gat1_wad, gat1_b,
         gat2_W, gat2_M, gat2_wad, gat2_b,
         s1_w, s1_b, s2_col, s2_b,
         delta_ref, Had_ref, pred_ref,
         C_scr, rd_scr, rg_scr):
    pid = pl.program_id(0)

    @pl.when(pid == 0)
    def _init():
        ids = jax.lax.broadcasted_iota(jnp.int32, (N, E), 0)
        s_oh = (ids == edge_ref[0:1, :]).astype(F32)
        d_oh = (ids == edge_ref[1:2, :]).astype(F32)
        C_T = jax.lax.dot_general(s_oh, d_oh, (((1,), (1,)), ((), ())),
                                  preferred_element_type=F32)
        ri = jax.lax.broadcasted_iota(jnp.int32, (N, N), 0)
        ci = jax.lax.broadcasted_iota(jnp.int32, (N, N), 1)
        C_scr[...] = C_T + (ri == ci).astype(F32)

        h = jnp.zeros((N, DR), F32)
        for t in range(K_SEQ):
            x_t = seq_ref[:, t:t + 1]
            r = jax.nn.sigmoid(x_t * wi_r[...] + bi_r[...]
                               + _dot(h, wh_r[...]) + bh_r[...])
            z = jax.nn.sigmoid(x_t * wi_z[...] + bi_z[...]
                               + _dot(h, wh_z[...]) + bh_z[...])
            nn_ = jnp.tanh(x_t * wi_n[...] + bi_n[...]
                           + r * (_dot(h, wh_n[...]) + bh_n[...]))
            h = (1.0 - z) * nn_ + z * h
        x = jnp.maximum(_dot(h, rl1_w[...]) + rl1_b[...], 0.0)
        rec = _dot(x, rl2_w[...]) + rl2_b[...]
        rd_scr[...] = _dot(rec, d1R[...]) + d1_b[...]
        rg_scr[...] = _dot(rec, g1R[...]) + g1_b[...]

    C_T = C_scr[...]
    H3 = H_ref[...]
    H2 = H3.reshape(SB * N, D)

    feat = feat_ref[...]
    cal = jnp.maximum(_dot(feat, cl1_w[...]) + cl1_b[...], 0.0)
    cal = _dot(cal, cl2_w[...]) + cl2_b[...]

    cal_d1 = _dot(cal, d1C[...])[:, None, :]
    cal_g1 = _dot(cal, g1C[...])[:, None, :]
    pre_d = _dot(H2, d1H[...]).reshape(SB, N, D) + rd_scr[...][None] + cal_d1
    pre_g = _dot(H2, g1H[...]).reshape(SB, N, D) + rg_scr[...][None] + cal_g1
    hid = jnp.maximum(pre_d, 0.0).reshape(SB * N, D)
    delta_raw = _dot(hid, d2_w[...]) + d2_b[...]
    gate = jax.nn.sigmoid(pre_g.reshape(SB * N, D))
    delta0 = delta_raw * gate

    delta0_3 = delta0.reshape(SB, N, D)
    xh0 = _dot(delta0, gat1_W01[..., :D]).reshape(SB, N, D)
    xh1 = _dot(delta0, gat1_W01[..., D:]).reshape(SB, N, D)
    p_v = _attn_p(delta0_3, gat1_M[...], gat1_wad[...], C_T, 0.5)
    x_v = jnp.concatenate([xh0, xh1], axis=1)
    agg1 = jax.lax.dot_general(p_v, x_v, (((1,), (1,)), ((0,), (0,))),
                               preferred_element_type=F32)
    x1 = jnp.maximum(agg1 + gat1_b[...][None], 0.0)

    xl2 = _dot(x1.reshape(SB * N, D), gat2_W[...]).reshape(SB, N, D)
    p2 = _attn_p(x1, gat2_M[...], gat2_wad[...], C_T, 1.0)
    o2 = jax.lax.dot_general(p2, xl2, (((1,), (1,)), ((0,), (0,))),
                             preferred_element_type=F32)
    delta = jnp.maximum(o2 + gat2_b[...][None], 0.0)

    H_ad = H3 + delta
    delta_ref[...] = delta
    Had_ref[...] = H_ad

    hh = jnp.maximum(_cmm(H_ad, s1_w[...]) + s1_b[...][None], 0.0)
    pred_ref[...] = _cmm(hh, s2_col[...]) + s2_b[...]


def kernel(H_base_bank, recent_speed_seq, edge_index, params):
    p = params
    H = H_base_bank.reshape(B, N, D)
    seq = recent_speed_seq[:, :, 0]

    wd = jnp.repeat(jnp.arange(7, dtype=F32), S_SLOTS)
    sl = jnp.tile(jnp.arange(S_SLOTS, dtype=F32), (7,))
    ta = (2.0 * math.pi / S_SLOTS) * sl
    wa = (2.0 * math.pi / 7.0) * wd
    cyc = jnp.stack([jnp.sin(ta), jnp.cos(ta), jnp.sin(wa), jnp.cos(wa)],
                    axis=-1)
    wk = ((wd == 5.0) | (wd == 6.0)).astype(F32)[:, None]
    feat = jnp.concatenate(
        [jnp.repeat(p['wd_emb'], S_SLOTS, axis=0),
         jnp.tile(p['slot_emb'], (7, 1)), cyc, wk], axis=-1)

    r2 = lambda a: a.reshape(1, -1)
    w_ih, w_hh = p['gru_W_ih'], p['gru_W_hh']
    b_ih, b_hh = p['gru_b_ih'], p['gru_b_hh']
    ins = [
        H, feat, seq, edge_index,
        w_ih[:DR].T, w_ih[DR:2 * DR].T, w_ih[2 * DR:].T,
        w_hh[:DR].T, w_hh[DR:2 * DR].T, w_hh[2 * DR:].T,
        r2(b_ih[:DR]), r2(b_ih[DR:2 * DR]), r2(b_ih[2 * DR:]),
        r2(b_hh[:DR]), r2(b_hh[DR:2 * DR]), r2(b_hh[2 * DR:]),
        p['rec_l1_w'], r2(p['rec_l1_b']), p['rec_l2_w'], r2(p['rec_l2_b']),
        p['cal_l1_w'], r2(p['cal_l1_b']), p['cal_l2_w'], r2(p['cal_l2_b']),
        p['d1_w'][:D], p['d1_w'][D:D + DR], p['d1_w'][D + DR:], r2(p['d1_b']),
        p['d2_w'], r2(p['d2_b']),
        p['g1_w'][:D], p['g1_w'][D:D + DR], p['g1_w'][D + DR:], r2(p['g1_b']),
        p['gat1_W'],
        jnp.concatenate(
            [jnp.broadcast_to((p['gat1_W'][:, h * D:(h + 1) * D]
                               @ p['gat1_as'][h])[:, None], (D, N))
             for h in (0, 1)], axis=1),
        jnp.stack([p['gat1_W'][:, :D] @ p['gat1_ad'][0],
                   p['gat1_W'][:, D:] @ p['gat1_ad'][1]], axis=1),
        r2(p['gat1_b']),
        p['gat2_W'],
        jnp.broadcast_to((p['gat2_W'] @ p['gat2_as'][0])[:, None], (D, N)),
        (p['gat2_W'] @ p['gat2_ad'][0])[:, None], r2(p['gat2_b']),
        p['s1_w'], r2(p['s1_b']), p['s2_w'], p['s2_b'].reshape(1, 1, 1),
    ]

    def const_spec(a):
        zeros = (0,) * a.ndim
        return pl.BlockSpec(a.shape, lambda i, z=zeros: z)

    in_specs = ([pl.BlockSpec((SB, N, D), lambda i: (i, 0, 0)),
                 pl.BlockSpec((SB, 29), lambda i: (i, 0))]
                + [const_spec(a) for a in ins[2:]])

    out_shape = (jax.ShapeDtypeStruct((B, N, D), F32),
                 jax.ShapeDtypeStruct((B, N, D), F32),
                 jax.ShapeDtypeStruct((B, N, 1), F32))
    out_specs = (pl.BlockSpec((SB, N, D), lambda i: (i, 0, 0)),
                 pl.BlockSpec((SB, N, D), lambda i: (i, 0, 0)),
                 pl.BlockSpec((SB, N, 1), lambda i: (i, 0, 0)))

    delta, H_ad, pred = pl.pallas_call(
        _krn,
        grid=(B // SB,),
        in_specs=in_specs,
        out_specs=out_specs,
        out_shape=out_shape,
        scratch_shapes=[pltpu.VMEM((N, N), F32), pltpu.VMEM((N, D), F32),
                        pltpu.VMEM((N, D), F32)],
        interpret=_INTERPRET,
    )(*ins)

    return (delta.reshape(W_DAYS, S_SLOTS, N, D),
            H_ad.reshape(W_DAYS, S_SLOTS, N, D),
            pred.reshape(W_DAYS, S_SLOTS, N))


# speedup vs baseline: 1.0792x; 1.0326x over previous
import math

import jax
import jax.numpy as jnp
from jax.experimental import pallas as pl
from jax.experimental.pallas import tpu as pltpu

W_DAYS, S_SLOTS = 7, 288
N, E = 64, 256
D, DR, DC, K_SEQ = 64, 32, 32, 12
B = W_DAYS * S_SLOTS
SB = 144
F32 = jnp.float32

_INTERPRET = False


BF16 = jnp.bfloat16


def _dot(a, b):
    return jax.lax.dot_general(a.astype(BF16), b.astype(BF16),
                               (((1,), (0,)), ((), ())),
                               preferred_element_type=F32)


def _cmm(a, b):
    return jax.lax.dot_general(a.astype(BF16), b.astype(BF16),
                               (((2,), (0,)), ((), ())),
                               preferred_element_type=F32)


def _attn_p(z, M, wad, C_rep, scale):
    nh = wad.shape[1]
    ad_rows = jnp.swapaxes(_cmm(z, wad), 1, 2)
    ps = []
    for h in range(nh):
        e = _cmm(z, M[:, h * N:(h + 1) * N])
        e = e + ad_rows[:, h:h + 1, :]
        e = jnp.maximum(e, 0.2 * e)
        p = C_rep[None] * jnp.exp(e)
        rden = scale / (jnp.sum(p, axis=1) + 1e-16)
        ps.append(p * rden[:, None, :])
    return jnp.concatenate(ps, axis=1) if nh > 1 else ps[0]


def _krn(H_ref, feat_ref, seq_ref, edge_ref,
         wi_r, wi_z, wi_n, wh_r, wh_z, wh_n,
         bi_r, bi_z, bi_n, bh_r, bh_z, bh_n,
         rl1_w, rl1_b, rl2_w, rl2_b,
         cl1_w, cl1_b, cl2_w, cl2_b,
         d1H, d1R, d1C, d1_b, d2_w, d2_b, g1H, g1R, g1C, g1_b,
         gat1_W01, gat1_M, gat1_wad, gat1_b,
         gat2_W, gat2_M, gat2_wad, gat2_b,
         s1_w, s1_b, s2_col, s2_b,
         delta_ref, Had_ref, pred_ref,
         C_scr, rd_scr, rg_scr):
    pid = pl.program_id(0)

    @pl.when(pid == 0)
    def _init():
        ids = jax.lax.broadcasted_iota(jnp.int32, (N, E), 0)
        s_oh = (ids == edge_ref[0:1, :]).astype(F32)
        d_oh = (ids == edge_ref[1:2, :]).astype(F32)
        C_T = jax.lax.dot_general(s_oh, d_oh, (((1,), (1,)), ((), ())),
                                  preferred_element_type=F32)
        ri = jax.lax.broadcasted_iota(jnp.int32, (N, N), 0)
        ci = jax.lax.broadcasted_iota(jnp.int32, (N, N), 1)
        C_scr[...] = C_T + (ri == ci).astype(F32)

        h = jnp.zeros((N, DR), F32)
        for t in range(K_SEQ):
            x_t = seq_ref[:, t:t + 1]
            r = jax.nn.sigmoid(x_t * wi_r[...] + bi_r[...]
                               + _dot(h, wh_r[...]) + bh_r[...])
            z = jax.nn.sigmoid(x_t * wi_z[...] + bi_z[...]
                               + _dot(h, wh_z[...]) + bh_z[...])
            nn_ = jnp.tanh(x_t * wi_n[...] + bi_n[...]
                           + r * (_dot(h, wh_n[...]) + bh_n[...]))
            h = (1.0 - z) * nn_ + z * h
        x = jnp.maximum(_dot(h, rl1_w[...]) + rl1_b[...], 0.0)
        rec = _dot(x, rl2_w[...]) + rl2_b[...]
        rd_scr[...] = _dot(rec, d1R[...]) + d1_b[...]
        rg_scr[...] = _dot(rec, g1R[...]) + g1_b[...]

    C_T = C_scr[...]
    H3 = H_ref[...]
    H2 = H3.reshape(SB * N, D)

    feat = feat_ref[...]
    cal = jnp.maximum(_dot(feat, cl1_w[...]) + cl1_b[...], 0.0)
    cal = _dot(cal, cl2_w[...]) + cl2_b[...]

    cal_d1 = _dot(cal, d1C[...])[:, None, :]
    cal_g1 = _dot(cal, g1C[...])[:, None, :]
    pre_d = _dot(H2, d1H[...]).reshape(SB, N, D) + rd_scr[...][None] + cal_d1
    pre_g = _dot(H2, g1H[...]).reshape(SB, N, D) + rg_scr[...][None] + cal_g1
    hid = jnp.maximum(pre_d, 0.0).reshape(SB * N, D)
    delta_raw = _dot(hid, d2_w[...]) + d2_b[...]
    gate = jax.nn.sigmoid(pre_g.reshape(SB * N, D))
    delta0 = delta_raw * gate

    delta0_3 = delta0.reshape(SB, N, D)
    xh0 = _dot(delta0, gat1_W01[..., :D]).reshape(SB, N, D)
    xh1 = _dot(delta0, gat1_W01[..., D:]).reshape(SB, N, D)
    p_v = _attn_p(delta0_3, gat1_M[...], gat1_wad[...], C_T, 0.5)
    x_v = jnp.concatenate([xh0, xh1], axis=1)
    agg1 = jax.lax.dot_general(p_v.astype(BF16), x_v.astype(BF16),
                               (((1,), (1,)), ((0,), (0,))),
                               preferred_element_type=F32)
    x1 = jnp.maximum(agg1 + gat1_b[...][None], 0.0)

    xl2 = _dot(x1.reshape(SB * N, D), gat2_W[...]).reshape(SB, N, D)
    p2 = _attn_p(x1, gat2_M[...], gat2_wad[...], C_T, 1.0)
    o2 = jax.lax.dot_general(p2.astype(BF16), xl2.astype(BF16),
                             (((1,), (1,)), ((0,), (0,))),
                             preferred_element_type=F32)
    delta = jnp.maximum(o2 + gat2_b[...][None], 0.0)

    H_ad = H3 + delta
    delta_ref[...] = delta
    Had_ref[...] = H_ad

    hh = jnp.maximum(_cmm(H_ad, s1_w[...]) + s1_b[...][None], 0.0)
    pred_ref[...] = _cmm(hh, s2_col[...]) + s2_b[...]


def kernel(H_base_bank, recent_speed_seq, edge_index, params):
    p = params
    H = H_base_bank.reshape(B, N, D)
    seq = recent_speed_seq[:, :, 0]

    wd = jnp.repeat(jnp.arange(7, dtype=F32), S_SLOTS)
    sl = jnp.tile(jnp.arange(S_SLOTS, dtype=F32), (7,))
    ta = (2.0 * math.pi / S_SLOTS) * sl
    wa = (2.0 * math.pi / 7.0) * wd
    cyc = jnp.stack([jnp.sin(ta), jnp.cos(ta), jnp.sin(wa), jnp.cos(wa)],
                    axis=-1)
    wk = ((wd == 5.0) | (wd == 6.0)).astype(F32)[:, None]
    feat = jnp.concatenate(
        [jnp.repeat(p['wd_emb'], S_SLOTS, axis=0),
         jnp.tile(p['slot_emb'], (7, 1)), cyc, wk], axis=-1)

    r2 = lambda a: a.reshape(1, -1)
    w_ih, w_hh = p['gru_W_ih'], p['gru_W_hh']
    b_ih, b_hh = p['gru_b_ih'], p['gru_b_hh']
    ins = [
        H, feat, seq, edge_index,
        w_ih[:DR].T, w_ih[DR:2 * DR].T, w_ih[2 * DR:].T,
        w_hh[:DR].T, w_hh[DR:2 * DR].T, w_hh[2 * DR:].T,
        r2(b_ih[:DR]), r2(b_ih[DR:2 * DR]), r2(b_ih[2 * DR:]),
        r2(b_hh[:DR]), r2(b_hh[DR:2 * DR]), r2(b_hh[2 * DR:]),
        p['rec_l1_w'], r2(p['rec_l1_b']), p['rec_l2_w'], r2(p['rec_l2_b']),
        p['cal_l1_w'], r2(p['cal_l1_b']), p['cal_l2_w'], r2(p['cal_l2_b']),
        p['d1_w'][:D], p['d1_w'][D:D + DR], p['d1_w'][D + DR:], r2(p['d1_b']),
        p['d2_w'], r2(p['d2_b']),
        p['g1_w'][:D], p['g1_w'][D:D + DR], p['g1_w'][D + DR:], r2(p['g1_b']),
        p['gat1_W'],
        jnp.concatenate(
            [jnp.broadcast_to((p['gat1_W'][:, h * D:(h + 1) * D]
                               @ p['gat1_as'][h])[:, None], (D, N))
             for h in (0, 1)], axis=1),
        jnp.stack([p['gat1_W'][:, :D] @ p['gat1_ad'][0],
                   p['gat1_W'][:, D:] @ p['gat1_ad'][1]], axis=1),
        r2(p['gat1_b']),
        p['gat2_W'],
        jnp.broadcast_to((p['gat2_W'] @ p['gat2_as'][0])[:, None], (D, N)),
        (p['gat2_W'] @ p['gat2_ad'][0])[:, None], r2(p['gat2_b']),
        p['s1_w'], r2(p['s1_b']), p['s2_w'], p['s2_b'].reshape(1, 1, 1),
    ]

    def const_spec(a):
        zeros = (0,) * a.ndim
        return pl.BlockSpec(a.shape, lambda i, z=zeros: z)

    in_specs = ([pl.BlockSpec((SB, N, D), lambda i: (i, 0, 0)),
                 pl.BlockSpec((SB, 29), lambda i: (i, 0))]
                + [const_spec(a) for a in ins[2:]])

    out_shape = (jax.ShapeDtypeStruct((B, N, D), F32),
                 jax.ShapeDtypeStruct((B, N, D), F32),
                 jax.ShapeDtypeStruct((B, N, 1), F32))
    out_specs = (pl.BlockSpec((SB, N, D), lambda i: (i, 0, 0)),
                 pl.BlockSpec((SB, N, D), lambda i: (i, 0, 0)),
                 pl.BlockSpec((SB, N, 1), lambda i: (i, 0, 0)))

    delta, H_ad, pred = pl.pallas_call(
        _krn,
        grid=(B // SB,),
        in_specs=in_specs,
        out_specs=out_specs,
        out_shape=out_shape,
        scratch_shapes=[pltpu.VMEM((N, N), F32), pltpu.VMEM((N, D), F32),
                        pltpu.VMEM((N, D), F32)],
        interpret=_INTERPRET,
    )(*ins)

    return (delta.reshape(W_DAYS, S_SLOTS, N, D),
            H_ad.reshape(W_DAYS, S_SLOTS, N, D),
            pred.reshape(W_DAYS, S_SLOTS, N))
